# baseline (speedup 1.0000x reference)
import jax
import jax.numpy as jnp
from jax import lax
from jax.experimental import pallas as pl
from jax.experimental.pallas import tpu as pltpu

XS = 256
HS = 512
RS = 64
MC = 1024
B = 512
FB = 1.0
TAU = 1.0


def _idx_body(x_ref, c_ref, wfc_ref, bfc_ref, u_ref, idx_ref):
    xc = jnp.concatenate([x_ref[...], c_ref[...]], axis=1)
    logits = jnp.dot(xc, wfc_ref[...], preferred_element_type=jnp.float32)
    logits = logits + bfc_ref[...]
    u = u_ref[...]
    gumbel = -jnp.log(1e-20 - jnp.log(1e-20 + u))
    s = (logits + gumbel) * TAU
    m = jnp.max(s, axis=1, keepdims=True)
    col = lax.broadcasted_iota(jnp.int32, s.shape, 1)
    big = jnp.where(s == m, col, jnp.int32(MC))
    idx_ref[...] = jnp.min(big, axis=1, keepdims=True)


_NCHUNK = 16
_CB = B // _NCHUNK


def _copy_gather_body(idx_ref, hm_ref, cp_ref, he_ref, sem_big, sem_row):
    for k in range(_NCHUNK):
        pltpu.make_async_copy(
            hm_ref.at[pl.ds(k * _CB, _CB)],
            cp_ref.at[pl.ds(k * _CB, _CB)], sem_big).start()

    def start(b, _):
        pltpu.make_async_copy(
            hm_ref.at[b, idx_ref[b]], he_ref.at[b], sem_row).start()
        return 0

    lax.fori_loop(0, B, start, 0)

    def drain(b, _):
        pltpu.make_async_copy(
            hm_ref.at[b, idx_ref[b]], he_ref.at[b], sem_row).wait()
        return 0

    lax.fori_loop(0, B, drain, 0)

    for k in range(_NCHUNK):
        pltpu.make_async_copy(
            hm_ref.at[pl.ds(k * _CB, _CB)],
            cp_ref.at[pl.ds(k * _CB, _CB)], sem_big).wait()


def _dense_body(x_ref, c_ref, he_ref, wf1_ref, b1_ref, wf_ref,
                b_ref, wt_ref, bt_ref, newr_ref, wv_ref):
    x = x_ref[...]
    c = c_ref[...]
    he = he_ref[...]
    concat = jnp.concatenate([x, c, he], axis=1)
    concat1 = jax.nn.sigmoid(
        jnp.dot(concat, wf1_ref[...], preferred_element_type=jnp.float32)
        + b1_ref[...])
    catm = jnp.concatenate([x, concat[:, XS:] * concat1], axis=1)
    gates = jnp.dot(catm, wf_ref[...], preferred_element_type=jnp.float32)
    gates = gates + b_ref[...]
    gi = gates[:, 0:HS]
    gj = gates[:, HS:2 * HS]
    gf = gates[:, 2 * HS:3 * HS]
    go = gates[:, 3 * HS:4 * HS]
    gom = gates[:, 4 * HS:4 * HS + RS]
    new_c = jnp.tanh(c * jax.nn.sigmoid(gf + FB)
                     + jax.nn.sigmoid(gi) * jnp.tanh(gj))
    new_h = new_c * jax.nn.sigmoid(go)
    r = he * jax.nn.sigmoid(gom)
    newr_ref[...] = jnp.concatenate([new_h, r], axis=1)
    wv_ref[...] = (jnp.dot(new_c, wt_ref[...], preferred_element_type=jnp.float32)
                   + bt_ref[...])


def _scatter_body(idx_ref, wv_ref, cp_ref, out_ref, sem):
    del cp_ref

    def start(b, _):
        pltpu.make_async_copy(
            wv_ref.at[b], out_ref.at[b, idx_ref[b]], sem).start()
        return 0

    lax.fori_loop(0, B, start, 0)

    def drain(b, _):
        pltpu.make_async_copy(
            wv_ref.at[b], out_ref.at[b, idx_ref[b]], sem).wait()
        return 0

    lax.fori_loop(0, B, drain, 0)


def kernel(x, c, hmem, u, W_full, bias, W_full1, bias1, W_fc, b_fc,
           W_trans, b_trans):
    idx_loc = pl.pallas_call(
        _idx_body,
        out_shape=jax.ShapeDtypeStruct((B, 1), jnp.int32),
    )(x, c, W_fc, b_fc.reshape(1, MC), u)

    cp, h_entry = pl.pallas_call(
        _copy_gather_body,
        in_specs=[
            pl.BlockSpec(memory_space=pltpu.SMEM),
            pl.BlockSpec(memory_space=pl.ANY),
        ],
        out_specs=[
            pl.BlockSpec(memory_space=pl.ANY),
            pl.BlockSpec(memory_space=pltpu.VMEM),
        ],
        out_shape=[
            jax.ShapeDtypeStruct((B, MC, RS), jnp.float32),
            jax.ShapeDtypeStruct((B, RS), jnp.float32),
        ],
        scratch_shapes=[pltpu.SemaphoreType.DMA, pltpu.SemaphoreType.DMA],
    )(idx_loc.reshape(B), hmem)

    new_r, write_val = pl.pallas_call(
        _dense_body,
        out_shape=[
            jax.ShapeDtypeStruct((B, HS + RS), jnp.float32),
            jax.ShapeDtypeStruct((B, RS), jnp.float32),
        ],
    )(x, c, h_entry, W_full1, bias1.reshape(1, -1), W_full,
      bias.reshape(1, -1), W_trans, b_trans.reshape(1, -1))

    new_hmem = pl.pallas_call(
        _scatter_body,
        in_specs=[
            pl.BlockSpec(memory_space=pltpu.SMEM),
            pl.BlockSpec(memory_space=pltpu.VMEM),
            pl.BlockSpec(memory_space=pl.ANY),
        ],
        out_specs=pl.BlockSpec(memory_space=pl.ANY),
        out_shape=jax.ShapeDtypeStruct((B, MC, RS), jnp.float32),
        scratch_shapes=[pltpu.SemaphoreType.DMA],
        input_output_aliases={2: 0},
    )(idx_loc.reshape(B), write_val, cp)

    return new_r, new_hmem


# speedup vs baseline: 15.2352x; 15.2352x over previous
import jax
import jax.numpy as jnp
from jax import lax
from jax.experimental import pallas as pl
from jax.experimental.pallas import tpu as pltpu

XS = 256
HS = 512
RS = 64
MC = 1024
B = 512
FB = 1.0
TAU = 1.0


def _idx_body(x_ref, c_ref, wfc_ref, bfc_ref, u_ref, idx_ref):
    xc = jnp.concatenate([x_ref[...], c_ref[...]], axis=1)
    logits = jnp.dot(xc, wfc_ref[...], preferred_element_type=jnp.float32)
    logits = logits + bfc_ref[...]
    u = u_ref[...]
    gumbel = -jnp.log(1e-20 - jnp.log(1e-20 + u))
    s = (logits + gumbel) * TAU
    m = jnp.max(s, axis=1, keepdims=True)
    col = lax.broadcasted_iota(jnp.int32, s.shape, 1)
    big = jnp.where(s == m, col, jnp.int32(MC))
    idx_ref[...] = jnp.min(big, axis=1, keepdims=True)


_BB = 16


def _stream_body(idx_ref, hm_ref, cp_ref, he_ref):
    blk = hm_ref[...]
    cp_ref[...] = blk
    slot = lax.broadcasted_iota(jnp.int32, (_BB, MC, 1), 1)
    hit = (slot == idx_ref[...][:, :, None]).astype(jnp.float32)
    he_ref[...] = jnp.sum(blk * hit, axis=1)


def _dense_body(x_ref, c_ref, he_ref, wf1_ref, b1_ref, wf_ref,
                b_ref, wt_ref, bt_ref, newr_ref, wv_ref):
    x = x_ref[...]
    c = c_ref[...]
    he = he_ref[...]
    concat = jnp.concatenate([x, c, he], axis=1)
    concat1 = jax.nn.sigmoid(
        jnp.dot(concat, wf1_ref[...], preferred_element_type=jnp.float32)
        + b1_ref[...])
    catm = jnp.concatenate([x, concat[:, XS:] * concat1], axis=1)
    gates = jnp.dot(catm, wf_ref[...], preferred_element_type=jnp.float32)
    gates = gates + b_ref[...]
    gi = gates[:, 0:HS]
    gj = gates[:, HS:2 * HS]
    gf = gates[:, 2 * HS:3 * HS]
    go = gates[:, 3 * HS:4 * HS]
    gom = gates[:, 4 * HS:4 * HS + RS]
    new_c = jnp.tanh(c * jax.nn.sigmoid(gf + FB)
                     + jax.nn.sigmoid(gi) * jnp.tanh(gj))
    new_h = new_c * jax.nn.sigmoid(go)
    r = he * jax.nn.sigmoid(gom)
    newr_ref[...] = jnp.concatenate([new_h, r], axis=1)
    wv_ref[...] = (jnp.dot(new_c, wt_ref[...], preferred_element_type=jnp.float32)
                   + bt_ref[...])


def _scatter_body(idx_ref, wv_ref, cp_ref, out_ref, sem):
    del cp_ref

    def start(b, _):
        pltpu.make_async_copy(
            wv_ref.at[b], out_ref.at[b, idx_ref[b]], sem).start()
        return 0

    lax.fori_loop(0, B, start, 0)

    def drain(b, _):
        pltpu.make_async_copy(
            wv_ref.at[b], out_ref.at[b, idx_ref[b]], sem).wait()
        return 0

    lax.fori_loop(0, B, drain, 0)


def kernel(x, c, hmem, u, W_full, bias, W_full1, bias1, W_fc, b_fc,
           W_trans, b_trans):
    idx_loc = pl.pallas_call(
        _idx_body,
        out_shape=jax.ShapeDtypeStruct((B, 1), jnp.int32),
    )(x, c, W_fc, b_fc.reshape(1, MC), u)

    cp, h_entry = pl.pallas_call(
        _stream_body,
        grid=(B // _BB,),
        in_specs=[
            pl.BlockSpec((_BB, 1), lambda i: (i, 0)),
            pl.BlockSpec((_BB, MC, RS), lambda i: (i, 0, 0)),
        ],
        out_specs=[
            pl.BlockSpec((_BB, MC, RS), lambda i: (i, 0, 0)),
            pl.BlockSpec((_BB, RS), lambda i: (i, 0)),
        ],
        out_shape=[
            jax.ShapeDtypeStruct((B, MC, RS), jnp.float32),
            jax.ShapeDtypeStruct((B, RS), jnp.float32),
        ],
        compiler_params=pltpu.CompilerParams(
            dimension_semantics=("arbitrary",)),
    )(idx_loc, hmem)

    new_r, write_val = pl.pallas_call(
        _dense_body,
        out_shape=[
            jax.ShapeDtypeStruct((B, HS + RS), jnp.float32),
            jax.ShapeDtypeStruct((B, RS), jnp.float32),
        ],
    )(x, c, h_entry, W_full1, bias1.reshape(1, -1), W_full,
      bias.reshape(1, -1), W_trans, b_trans.reshape(1, -1))

    new_hmem = pl.pallas_call(
        _scatter_body,
        in_specs=[
            pl.BlockSpec(memory_space=pltpu.SMEM),
            pl.BlockSpec(memory_space=pltpu.VMEM),
            pl.BlockSpec(memory_space=pl.ANY),
        ],
        out_specs=pl.BlockSpec(memory_space=pl.ANY),
        out_shape=jax.ShapeDtypeStruct((B, MC, RS), jnp.float32),
        scratch_shapes=[pltpu.SemaphoreType.DMA],
        input_output_aliases={2: 0},
    )(idx_loc.reshape(B), write_val, cp)

    return new_r, new_hmem


# speedup vs baseline: 23.2712x; 1.5275x over previous
import jax
import jax.numpy as jnp
from jax import lax
from jax.experimental import pallas as pl
from jax.experimental.pallas import tpu as pltpu

XS = 256
HS = 512
RS = 64
MC = 1024
B = 512
FB = 1.0
TAU = 1.0
LW = 128


def _idx_body(x_ref, c_ref, wfc_ref, bfc_ref, u_ref, idx_ref):
    xc = jnp.concatenate([x_ref[...], c_ref[...]], axis=1)
    logits = jnp.dot(xc, wfc_ref[...], preferred_element_type=jnp.float32)
    logits = logits + bfc_ref[...]
    u = u_ref[...]
    gumbel = -jnp.log(1e-20 - jnp.log(1e-20 + u))
    s = (logits + gumbel) * TAU
    m = jnp.max(s, axis=1, keepdims=True)
    col = lax.broadcasted_iota(jnp.int32, s.shape, 1)
    big = jnp.where(s == m, col, jnp.int32(MC))
    idx_ref[...] = jnp.min(big, axis=1, keepdims=True)


def _gather_body(idxpf_ref, win_ref, he_ref):
    b = pl.program_id(0)
    lane = idxpf_ref[b] % LW
    win = win_ref[0]
    col = lax.broadcasted_iota(jnp.int32, (RS, LW), 1)
    hit = (col == lane).astype(jnp.float32)
    he_ref[...] = jnp.sum(win * hit, axis=1)[None, None, :]


def _dense_body(x_ref, c_ref, he_ref, wf1_ref, b1_ref, wf_ref,
                b_ref, wt_ref, bt_ref, newr_ref, wv_ref):
    x = x_ref[...]
    c = c_ref[...]
    he = he_ref[...]
    concat = jnp.concatenate([x, c, he], axis=1)
    concat1 = jax.nn.sigmoid(
        jnp.dot(concat, wf1_ref[...], preferred_element_type=jnp.float32)
        + b1_ref[...])
    catm = jnp.concatenate([x, concat[:, XS:] * concat1], axis=1)
    gates = jnp.dot(catm, wf_ref[...], preferred_element_type=jnp.float32)
    gates = gates + b_ref[...]
    gi = gates[:, 0:HS]
    gj = gates[:, HS:2 * HS]
    gf = gates[:, 2 * HS:3 * HS]
    go = gates[:, 3 * HS:4 * HS]
    gom = gates[:, 4 * HS:4 * HS + RS]
    new_c = jnp.tanh(c * jax.nn.sigmoid(gf + FB)
                     + jax.nn.sigmoid(gi) * jnp.tanh(gj))
    new_h = new_c * jax.nn.sigmoid(go)
    r = he * jax.nn.sigmoid(gom)
    newr_ref[...] = jnp.concatenate([new_h, r], axis=1)
    wv_ref[...] = (jnp.dot(new_c, wt_ref[...], preferred_element_type=jnp.float32)
                   + bt_ref[...])


_BB = 16


def _stream_body(idx_ref, wv_ref, hm_ref, out_ref):
    blk = hm_ref[...]
    slot = lax.broadcasted_iota(jnp.int32, (_BB, 1, MC), 2)
    mask = slot == idx_ref[...][:, :, None]
    wv3 = wv_ref[...][:, :, None]
    out_ref[...] = jnp.where(mask, wv3, blk)


def kernel(x, c, hmem, u, W_full, bias, W_full1, bias1, W_fc, b_fc,
           W_trans, b_trans):
    hm_t = hmem.transpose(0, 2, 1)

    idx_loc = pl.pallas_call(
        _idx_body,
        out_shape=jax.ShapeDtypeStruct((B, 1), jnp.int32),
    )(x, c, W_fc, b_fc.reshape(1, MC), u)

    idx_flat = idx_loc.reshape(B)

    h_entry = pl.pallas_call(
        _gather_body,
        grid_spec=pltpu.PrefetchScalarGridSpec(
            num_scalar_prefetch=1,
            grid=(B,),
            in_specs=[
                pl.BlockSpec((1, RS, LW), lambda b, idx: (b, 0, idx[b] // LW)),
            ],
            out_specs=pl.BlockSpec((1, 1, RS), lambda b, idx: (b, 0, 0)),
        ),
        out_shape=jax.ShapeDtypeStruct((B, 1, RS), jnp.float32),
    )(idx_flat, hm_t)
    h_entry = h_entry.reshape(B, RS)

    new_r, write_val = pl.pallas_call(
        _dense_body,
        out_shape=[
            jax.ShapeDtypeStruct((B, HS + RS), jnp.float32),
            jax.ShapeDtypeStruct((B, RS), jnp.float32),
        ],
    )(x, c, h_entry, W_full1, bias1.reshape(1, -1), W_full,
      bias.reshape(1, -1), W_trans, b_trans.reshape(1, -1))

    out_t = pl.pallas_call(
        _stream_body,
        grid=(B // _BB,),
        in_specs=[
            pl.BlockSpec((_BB, 1), lambda i: (i, 0)),
            pl.BlockSpec((_BB, RS), lambda i: (i, 0)),
            pl.BlockSpec((_BB, RS, MC), lambda i: (i, 0, 0)),
        ],
        out_specs=pl.BlockSpec((_BB, RS, MC), lambda i: (i, 0, 0)),
        out_shape=jax.ShapeDtypeStruct((B, RS, MC), jnp.float32),
        compiler_params=pltpu.CompilerParams(
            dimension_semantics=("arbitrary",)),
    )(idx_loc, write_val, hm_t)

    return new_r, out_t.transpose(0, 2, 1)


# speedup vs baseline: 55.3137x; 2.3769x over previous
import jax
import jax.numpy as jnp
from jax import lax
from jax.experimental import pallas as pl
from jax.experimental.pallas import tpu as pltpu

XS = 256
HS = 512
RS = 64
MC = 1024
B = 512
FB = 1.0
TAU = 1.0
LW = 128


def _idx_body(x_ref, c_ref, wfc_ref, bfc_ref, u_ref, idx_ref):
    xc = jnp.concatenate([x_ref[...], c_ref[...]], axis=1)
    logits = jnp.dot(xc, wfc_ref[...], preferred_element_type=jnp.float32)
    logits = logits + bfc_ref[...]
    u = u_ref[...]
    gumbel = -jnp.log(1e-20 - jnp.log(1e-20 + u))
    s = (logits + gumbel) * TAU
    m = jnp.max(s, axis=1, keepdims=True)
    col = lax.broadcasted_iota(jnp.int32, s.shape, 1)
    big = jnp.where(s == m, col, jnp.int32(MC))
    idx_ref[...] = jnp.min(big, axis=1, keepdims=True)


def _gather_body(idxpf_ref, win_ref, he_ref):
    b = pl.program_id(0)
    lane = idxpf_ref[b] % LW
    win = win_ref[0]
    col = lax.broadcasted_iota(jnp.int32, (RS, LW), 1)
    hit = (col == lane).astype(jnp.float32)
    he_ref[...] = jnp.sum(win * hit, axis=1)[None, None, :]


def _dense_body(x_ref, c_ref, he_ref, wf1_ref, b1_ref, wf_ref,
                b_ref, wt_ref, bt_ref, newr_ref, wv_ref):
    x = x_ref[...]
    c = c_ref[...]
    he = he_ref[...]
    concat = jnp.concatenate([x, c, he], axis=1)
    concat1 = jax.nn.sigmoid(
        jnp.dot(concat, wf1_ref[...], preferred_element_type=jnp.float32)
        + b1_ref[...])
    catm = jnp.concatenate([x, concat[:, XS:] * concat1], axis=1)
    gates = jnp.dot(catm, wf_ref[...], preferred_element_type=jnp.float32)
    gates = gates + b_ref[...]
    gi = gates[:, 0:HS]
    gj = gates[:, HS:2 * HS]
    gf = gates[:, 2 * HS:3 * HS]
    go = gates[:, 3 * HS:4 * HS]
    gom = gates[:, 4 * HS:4 * HS + RS]
    new_c = jnp.tanh(c * jax.nn.sigmoid(gf + FB)
                     + jax.nn.sigmoid(gi) * jnp.tanh(gj))
    new_h = new_c * jax.nn.sigmoid(go)
    r = he * jax.nn.sigmoid(gom)
    newr_ref[...] = jnp.concatenate([new_h, r], axis=1)
    wv_ref[...] = (jnp.dot(new_c, wt_ref[...], preferred_element_type=jnp.float32)
                   + bt_ref[...])


_BB = 16


def _stream_body(idx_ref, wv_ref, hm_ref, out_ref):
    blk = hm_ref[...]
    slot = lax.broadcasted_iota(jnp.int32, (_BB, 1, MC), 2)
    mask = slot == idx_ref[...][:, :, None]
    wv3 = wv_ref[...][:, :, None]
    out_ref[...] = jnp.where(mask, wv3, blk)


def kernel(x, c, hmem, u, W_full, bias, W_full1, bias1, W_fc, b_fc,
           W_trans, b_trans):
    hm_t = hmem.transpose(0, 2, 1)

    idx_loc = pl.pallas_call(
        _idx_body,
        out_shape=jax.ShapeDtypeStruct((B, 1), jnp.int32),
    )(x, c, W_fc, b_fc.reshape(1, MC), u)

    idx_flat = idx_loc.reshape(B)

    onehot = (jnp.arange(MC, dtype=jnp.int32)[None, :] == idx_loc).astype(jnp.float32)
    h_entry = jnp.einsum('brm,bm->br', hm_t, onehot)

    new_r, write_val = pl.pallas_call(
        _dense_body,
        out_shape=[
            jax.ShapeDtypeStruct((B, HS + RS), jnp.float32),
            jax.ShapeDtypeStruct((B, RS), jnp.float32),
        ],
    )(x, c, h_entry, W_full1, bias1.reshape(1, -1), W_full,
      bias.reshape(1, -1), W_trans, b_trans.reshape(1, -1))

    out_t = pl.pallas_call(
        _stream_body,
        grid=(B // _BB,),
        in_specs=[
            pl.BlockSpec((_BB, 1), lambda i: (i, 0)),
            pl.BlockSpec((_BB, RS), lambda i: (i, 0)),
            pl.BlockSpec((_BB, RS, MC), lambda i: (i, 0, 0)),
        ],
        out_specs=pl.BlockSpec((_BB, RS, MC), lambda i: (i, 0, 0)),
        out_shape=jax.ShapeDtypeStruct((B, RS, MC), jnp.float32),
        compiler_params=pltpu.CompilerParams(
            dimension_semantics=("arbitrary",)),
    )(idx_loc, write_val, hm_t)

    return new_r, out_t.transpose(0, 2, 1)


# speedup vs baseline: 67.3667x; 1.2179x over previous
import jax
import jax.numpy as jnp
from jax import lax
from jax.experimental import pallas as pl
from jax.experimental.pallas import tpu as pltpu

XS = 256
HS = 512
RS = 64
MC = 1024
B = 512
FB = 1.0
TAU = 1.0
LW = 128


def _idx_body(x_ref, c_ref, wfc_ref, bfc_ref, u_ref, idx_ref):
    xc = jnp.concatenate([x_ref[...], c_ref[...]], axis=1)
    logits = jnp.dot(xc, wfc_ref[...], preferred_element_type=jnp.float32)
    logits = logits + bfc_ref[...]
    u = u_ref[...]
    gumbel = -jnp.log(1e-20 - jnp.log(1e-20 + u))
    s = (logits + gumbel) * TAU
    m = jnp.max(s, axis=1, keepdims=True)
    col = lax.broadcasted_iota(jnp.int32, s.shape, 1)
    big = jnp.where(s == m, col, jnp.int32(MC))
    idx_ref[...] = jnp.min(big, axis=1, keepdims=True)


def _gather_body(idx_s, idx_v, hm_ref, he_ref, win, sem):
    def start(b, _):
        t = pl.multiple_of((idx_s[b] // LW) * LW, LW)
        pltpu.make_async_copy(
            hm_ref.at[b, :, pl.ds(t, LW)], win.at[b], sem).start()
        return 0

    lax.fori_loop(0, B, start, 0)

    def drain(b, _):
        t = pl.multiple_of((idx_s[b] // LW) * LW, LW)
        pltpu.make_async_copy(
            hm_ref.at[b, :, pl.ds(t, LW)], win.at[b], sem).wait()
        return 0

    lax.fori_loop(0, B, drain, 0)

    lane = lax.broadcasted_iota(jnp.int32, (B, 1, LW), 2)
    hit = (lane == (idx_v[...] % LW)[:, :, None]).astype(jnp.float32)
    he_ref[...] = jnp.sum(win[...] * hit, axis=2)


def _dense_body(x_ref, c_ref, he_ref, wf1_ref, b1_ref, wf_ref,
                b_ref, wt_ref, bt_ref, newr_ref, wv_ref):
    x = x_ref[...]
    c = c_ref[...]
    he = he_ref[...]
    concat = jnp.concatenate([x, c, he], axis=1)
    concat1 = jax.nn.sigmoid(
        jnp.dot(concat, wf1_ref[...], preferred_element_type=jnp.float32)
        + b1_ref[...])
    catm = jnp.concatenate([x, concat[:, XS:] * concat1], axis=1)
    gates = jnp.dot(catm, wf_ref[...], preferred_element_type=jnp.float32)
    gates = gates + b_ref[...]
    gi = gates[:, 0:HS]
    gj = gates[:, HS:2 * HS]
    gf = gates[:, 2 * HS:3 * HS]
    go = gates[:, 3 * HS:4 * HS]
    gom = gates[:, 4 * HS:4 * HS + RS]
    new_c = jnp.tanh(c * jax.nn.sigmoid(gf + FB)
                     + jax.nn.sigmoid(gi) * jnp.tanh(gj))
    new_h = new_c * jax.nn.sigmoid(go)
    r = he * jax.nn.sigmoid(gom)
    newr_ref[...] = jnp.concatenate([new_h, r], axis=1)
    wv_ref[...] = (jnp.dot(new_c, wt_ref[...], preferred_element_type=jnp.float32)
                   + bt_ref[...])


_BB = 16


def _stream_body(idx_ref, wv_ref, hm_ref, out_ref):
    blk = hm_ref[...]
    slot = lax.broadcasted_iota(jnp.int32, (_BB, 1, MC), 2)
    mask = slot == idx_ref[...][:, :, None]
    wv3 = wv_ref[...][:, :, None]
    out_ref[...] = jnp.where(mask, wv3, blk)


def kernel(x, c, hmem, u, W_full, bias, W_full1, bias1, W_fc, b_fc,
           W_trans, b_trans):
    hm_t = hmem.transpose(0, 2, 1)

    idx_loc = pl.pallas_call(
        _idx_body,
        out_shape=jax.ShapeDtypeStruct((B, 1), jnp.int32),
    )(x, c, W_fc, b_fc.reshape(1, MC), u)

    h_entry = pl.pallas_call(
        _gather_body,
        in_specs=[
            pl.BlockSpec(memory_space=pltpu.SMEM),
            pl.BlockSpec(memory_space=pltpu.VMEM),
            pl.BlockSpec(memory_space=pl.ANY),
        ],
        out_specs=pl.BlockSpec(memory_space=pltpu.VMEM),
        out_shape=jax.ShapeDtypeStruct((B, RS), jnp.float32),
        scratch_shapes=[
            pltpu.VMEM((B, RS, LW), jnp.float32),
            pltpu.SemaphoreType.DMA,
        ],
    )(idx_loc.reshape(B), idx_loc, hm_t)

    new_r, write_val = pl.pallas_call(
        _dense_body,
        out_shape=[
            jax.ShapeDtypeStruct((B, HS + RS), jnp.float32),
            jax.ShapeDtypeStruct((B, RS), jnp.float32),
        ],
    )(x, c, h_entry, W_full1, bias1.reshape(1, -1), W_full,
      bias.reshape(1, -1), W_trans, b_trans.reshape(1, -1))

    out_t = pl.pallas_call(
        _stream_body,
        grid=(B // _BB,),
        in_specs=[
            pl.BlockSpec((_BB, 1), lambda i: (i, 0)),
            pl.BlockSpec((_BB, RS), lambda i: (i, 0)),
            pl.BlockSpec((_BB, RS, MC), lambda i: (i, 0, 0)),
        ],
        out_specs=pl.BlockSpec((_BB, RS, MC), lambda i: (i, 0, 0)),
        out_shape=jax.ShapeDtypeStruct((B, RS, MC), jnp.float32),
        compiler_params=pltpu.CompilerParams(
            dimension_semantics=("arbitrary",)),
    )(idx_loc, write_val, hm_t)

    return new_r, out_t.transpose(0, 2, 1)


# speedup vs baseline: 68.3016x; 1.0139x over previous
import jax
import jax.numpy as jnp
from jax import lax
from jax.experimental import pallas as pl
from jax.experimental.pallas import tpu as pltpu

XS = 256
HS = 512
RS = 64
MC = 1024
B = 512
FB = 1.0
TAU = 1.0
LW = 128


def _idx_body(x_ref, c_ref, wfc_ref, bfc_ref, u_ref, idx_ref):
    xc = jnp.concatenate([x_ref[...], c_ref[...]], axis=1)
    logits = jnp.dot(xc, wfc_ref[...], preferred_element_type=jnp.float32)
    logits = logits + bfc_ref[...]
    u = u_ref[...]
    gumbel = -jnp.log(1e-20 - jnp.log(1e-20 + u))
    s = (logits + gumbel) * TAU
    m = jnp.max(s, axis=1, keepdims=True)
    col = lax.broadcasted_iota(jnp.int32, s.shape, 1)
    big = jnp.where(s == m, col, jnp.int32(MC))
    idx_ref[...] = jnp.min(big, axis=1, keepdims=True)


def _gather_body(idx_s, idx_v, hm_ref, he_ref, win, sem):
    def start(b, _):
        t = pl.multiple_of((idx_s[b] // LW) * LW, LW)
        pltpu.make_async_copy(
            hm_ref.at[b, :, pl.ds(t, LW)], win.at[b], sem).start()
        return 0

    lax.fori_loop(0, B, start, 0)

    def drain(b, _):
        t = pl.multiple_of((idx_s[b] // LW) * LW, LW)
        pltpu.make_async_copy(
            hm_ref.at[b, :, pl.ds(t, LW)], win.at[b], sem).wait()
        return 0

    lax.fori_loop(0, B, drain, 0)

    lane = lax.broadcasted_iota(jnp.int32, (B, 1, LW), 2)
    hit = (lane == (idx_v[...] % LW)[:, :, None]).astype(jnp.float32)
    he_ref[...] = jnp.sum(win[...] * hit, axis=2)


def _dense_body(x_ref, c_ref, he_ref, wf1_ref, b1_ref, wf_ref,
                b_ref, wt_ref, bt_ref, newr_ref, wv_ref):
    x = x_ref[...]
    c = c_ref[...]
    he = he_ref[...]
    concat = jnp.concatenate([x, c, he], axis=1)
    concat1 = jax.nn.sigmoid(
        jnp.dot(concat, wf1_ref[...], preferred_element_type=jnp.float32)
        + b1_ref[...])
    catm = jnp.concatenate([x, concat[:, XS:] * concat1], axis=1)
    gates = jnp.dot(catm, wf_ref[...], preferred_element_type=jnp.float32)
    gates = gates + b_ref[...]
    gi = gates[:, 0:HS]
    gj = gates[:, HS:2 * HS]
    gf = gates[:, 2 * HS:3 * HS]
    go = gates[:, 3 * HS:4 * HS]
    gom = gates[:, 4 * HS:4 * HS + RS]
    new_c = jnp.tanh(c * jax.nn.sigmoid(gf + FB)
                     + jax.nn.sigmoid(gi) * jnp.tanh(gj))
    new_h = new_c * jax.nn.sigmoid(go)
    r = he * jax.nn.sigmoid(gom)
    newr_ref[...] = jnp.concatenate([new_h, r], axis=1)
    wv_ref[...] = (jnp.dot(new_c, wt_ref[...], preferred_element_type=jnp.float32)
                   + bt_ref[...])


_BB = 32


def _stream_body(idx_ref, wv_ref, hm_ref, out_ref):
    blk = hm_ref[...]
    slot = lax.broadcasted_iota(jnp.int32, (_BB, 1, MC), 2)
    mask = slot == idx_ref[...][:, :, None]
    wv3 = wv_ref[...][:, :, None]
    out_ref[...] = jnp.where(mask, wv3, blk)


def kernel(x, c, hmem, u, W_full, bias, W_full1, bias1, W_fc, b_fc,
           W_trans, b_trans):
    hm_t = hmem.transpose(0, 2, 1)

    idx_loc = pl.pallas_call(
        _idx_body,
        out_shape=jax.ShapeDtypeStruct((B, 1), jnp.int32),
    )(x, c, W_fc, b_fc.reshape(1, MC), u)

    h_entry = pl.pallas_call(
        _gather_body,
        in_specs=[
            pl.BlockSpec(memory_space=pltpu.SMEM),
            pl.BlockSpec(memory_space=pltpu.VMEM),
            pl.BlockSpec(memory_space=pl.ANY),
        ],
        out_specs=pl.BlockSpec(memory_space=pltpu.VMEM),
        out_shape=jax.ShapeDtypeStruct((B, RS), jnp.float32),
        scratch_shapes=[
            pltpu.VMEM((B, RS, LW), jnp.float32),
            pltpu.SemaphoreType.DMA,
        ],
    )(idx_loc.reshape(B), idx_loc, hm_t)

    new_r, write_val = pl.pallas_call(
        _dense_body,
        out_shape=[
            jax.ShapeDtypeStruct((B, HS + RS), jnp.float32),
            jax.ShapeDtypeStruct((B, RS), jnp.float32),
        ],
    )(x, c, h_entry, W_full1, bias1.reshape(1, -1), W_full,
      bias.reshape(1, -1), W_trans, b_trans.reshape(1, -1))

    out_t = pl.pallas_call(
        _stream_body,
        grid=(B // _BB,),
        in_specs=[
            pl.BlockSpec((_BB, 1), lambda i: (i, 0)),
            pl.BlockSpec((_BB, RS), lambda i: (i, 0)),
            pl.BlockSpec((_BB, RS, MC), lambda i: (i, 0, 0)),
        ],
        out_specs=pl.BlockSpec((_BB, RS, MC), lambda i: (i, 0, 0)),
        out_shape=jax.ShapeDtypeStruct((B, RS, MC), jnp.float32),
        compiler_params=pltpu.CompilerParams(
            dimension_semantics=("arbitrary",)),
    )(idx_loc, write_val, hm_t)

    return new_r, out_t.transpose(0, 2, 1)
